# baseline (speedup 1.0000x reference)
import functools

import jax
import jax.numpy as jnp
from jax import lax
from jax.experimental import pallas as pl
from jax.experimental.pallas import tpu as pltpu
from jax.experimental.pallas import tpu_sc as plsc

_NC = 2
_NS = 16
_NW = _NC * _NS
_L = 16


def _coef_body(wt_ref, c_ref):
    wt = wt_ref[...]
    m = jnp.max(wt, axis=0, keepdims=True)
    e = jnp.exp(wt - m)
    s = e / jnp.sum(e, axis=0, keepdims=True)
    r = [s[i : i + 1, :] for i in range(14)]
    c0 = r[7] + r[8] + r[9] + r[10] + r[11] + r[12] + r[13]
    c1 = r[1] + r[2] + r[5] + r[6] - r[7] - r[8] - r[11] - r[12]
    c2 = r[3] + r[4] + r[5] + r[6] - r[7] - r[8] - r[9] - r[10]
    c3 = (r[0] - r[1] - r[3] - 2.0 * r[5] - r[6] + r[7] + 2.0 * r[8]
          + r[10] + r[12] - r[13])
    c_ref[...] = jnp.concatenate([c0, c1, c2, c3], axis=0)


def _fold_coefs(weights):
    out_dim = weights.shape[0]
    return pl.pallas_call(
        _coef_body,
        out_shape=jax.ShapeDtypeStruct((4, out_dim), jnp.float32),
    )(weights.T)


def _make_sc_kernel(batch, in_dim, out_dim):
    rows_per_w = batch // _NW
    chunks = out_dim // _L
    mesh = plsc.VectorSubcoreMesh(
        core_axis_name="c", subcore_axis_name="s",
        num_cores=_NC, num_subcores=_NS)

    @functools.partial(
        pl.kernel,
        mesh=mesh,
        compiler_params=pltpu.CompilerParams(needs_layout_passes=False),
        out_type=jax.ShapeDtypeStruct((batch, out_dim), jnp.float32),
        scratch_types=[
            pltpu.VMEM((out_dim,), jnp.int32),
            pltpu.VMEM((out_dim,), jnp.int32),
            pltpu.VMEM((4, out_dim), jnp.float32),
            pltpu.VMEM((in_dim,), jnp.float32),
            pltpu.VMEM((out_dim,), jnp.float32),
        ],
    )
    def sc_kernel(x_hbm, c_hbm, ia_hbm, ib_hbm, out_hbm,
                  ia_v, ib_v, c_v, x_v, o_v):
        wid = lax.axis_index("s") * _NC + lax.axis_index("c")
        base = wid * rows_per_w
        pltpu.sync_copy(ia_hbm, ia_v)
        pltpu.sync_copy(ib_hbm, ib_v)
        pltpu.sync_copy(c_hbm, c_v)

        def row_body(i, carry):
            row = base + i
            pltpu.sync_copy(x_hbm.at[row], x_v)

            def chunk_body(jc, carry2):
                off = jc * _L
                ja = ia_v[pl.ds(off, _L)]
                jb = ib_v[pl.ds(off, _L)]
                a = plsc.load_gather(x_v, [ja])
                b = plsc.load_gather(x_v, [jb])
                c0 = c_v[0, pl.ds(off, _L)]
                c1 = c_v[1, pl.ds(off, _L)]
                c2 = c_v[2, pl.ds(off, _L)]
                c3 = c_v[3, pl.ds(off, _L)]
                o_v[pl.ds(off, _L)] = c0 + a * c1 + b * c2 + (a * b) * c3
                return carry2

            lax.fori_loop(0, chunks, chunk_body, 0)
            pltpu.sync_copy(o_v, out_hbm.at[row])
            return carry

        lax.fori_loop(0, rows_per_w, row_body, 0)

    return sc_kernel


def kernel(x, weights, idx_a, idx_b):
    batch, in_dim = x.shape
    out_dim = weights.shape[0]
    c = _fold_coefs(weights)
    sc = _make_sc_kernel(batch, in_dim, out_dim)
    return sc(x, c, idx_a.astype(jnp.int32), idx_b.astype(jnp.int32))


# speedup vs baseline: 3.9887x; 3.9887x over previous
import functools

import jax
import jax.numpy as jnp
from jax import lax
from jax.experimental import pallas as pl
from jax.experimental.pallas import tpu as pltpu
from jax.experimental.pallas import tpu_sc as plsc

_NC = 2
_NS = 16
_NW = _NC * _NS
_L = 16
_R = 2
_IDX_BITS = 13


def _prep_body(wt_ref, ia_ref, ib_ref, c_ref, pk_ref):
    wt = wt_ref[...]
    m = jnp.max(wt, axis=0, keepdims=True)
    e = jnp.exp(wt - m)
    s = e / jnp.sum(e, axis=0, keepdims=True)
    r = [s[i : i + 1, :] for i in range(14)]
    c0 = r[7] + r[8] + r[9] + r[10] + r[11] + r[12] + r[13]
    c1 = r[1] + r[2] + r[5] + r[6] - r[7] - r[8] - r[11] - r[12]
    c2 = r[3] + r[4] + r[5] + r[6] - r[7] - r[8] - r[9] - r[10]
    c3 = (r[0] - r[1] - r[3] - 2.0 * r[5] - r[6] + r[7] + 2.0 * r[8]
          + r[10] + r[12] - r[13])
    c_ref[...] = jnp.concatenate([c0, c1, c2, c3], axis=0)
    pk_ref[...] = ia_ref[...] | (ib_ref[...] << _IDX_BITS)


def _prep(weights, idx_a, idx_b):
    out_dim = weights.shape[0]
    return pl.pallas_call(
        _prep_body,
        out_shape=[
            jax.ShapeDtypeStruct((4, out_dim), jnp.float32),
            jax.ShapeDtypeStruct((out_dim,), jnp.int32),
        ],
    )(weights.T, idx_a, idx_b)


def _make_sc_kernel(batch, in_dim, out_dim):
    rows_per_w = batch // _NW
    ngroups = rows_per_w // _R
    chunks = out_dim // _L
    mesh = plsc.VectorSubcoreMesh(
        core_axis_name="c", subcore_axis_name="s",
        num_cores=_NC, num_subcores=_NS)

    @functools.partial(
        pl.kernel,
        mesh=mesh,
        compiler_params=pltpu.CompilerParams(needs_layout_passes=False),
        out_type=jax.ShapeDtypeStruct((batch, out_dim), jnp.float32),
        scratch_types=[
            pltpu.VMEM((out_dim,), jnp.int32),
            pltpu.VMEM((4, out_dim), jnp.float32),
            pltpu.VMEM((_R, in_dim), jnp.float32),
            pltpu.VMEM((_R, in_dim), jnp.float32),
            pltpu.VMEM((_R, out_dim), jnp.float32),
            pltpu.VMEM((_R, out_dim), jnp.float32),
            pltpu.SemaphoreType.DMA,
            pltpu.SemaphoreType.DMA,
            pltpu.SemaphoreType.DMA,
            pltpu.SemaphoreType.DMA,
        ],
    )
    def sc_kernel(x_hbm, c_hbm, pk_hbm, out_hbm,
                  pk_v, c_v, x0_v, x1_v, o0_v, o1_v,
                  xs0, xs1, os0, os1):
        wid = lax.axis_index("s") * _NC + lax.axis_index("c")
        base = wid * rows_per_w
        pltpu.sync_copy(pk_hbm, pk_v)
        pltpu.sync_copy(c_hbm, c_v)

        xbufs = (x0_v, x1_v)
        xsems = (xs0, xs1)
        obufs = (o0_v, o1_v)
        osems = (os0, os1)

        def x_dma(g, phase):
            return pltpu.make_async_copy(
                x_hbm.at[pl.ds(base + g * _R, _R)], xbufs[phase], xsems[phase])

        def o_dma(g, phase):
            return pltpu.make_async_copy(
                obufs[phase], out_hbm.at[pl.ds(base + g * _R, _R)],
                osems[phase])

        x_dma(0, 0).start()

        def outer(i, carry):
            go = i * 2
            for phase in range(2):
                g = go + phase
                xbuf = xbufs[phase]
                obuf = obufs[phase]
                x_dma(g, phase).wait()

                @pl.when(g + 1 < ngroups)
                def _():
                    x_dma(g + 1, 1 - phase).start()

                @pl.when(g >= 2)
                def _():
                    o_dma(g - 2, phase).wait()

                @plsc.parallel_loop(0, chunks, unroll=4)
                def chunk(jc):
                    off = jc * _L
                    pv = pk_v[pl.ds(off, _L)]
                    ja = lax.bitwise_and(pv, (1 << _IDX_BITS) - 1)
                    jb = lax.shift_right_logical(pv, _IDX_BITS)
                    c0 = c_v[0, pl.ds(off, _L)]
                    c1 = c_v[1, pl.ds(off, _L)]
                    c2 = c_v[2, pl.ds(off, _L)]
                    c3 = c_v[3, pl.ds(off, _L)]
                    for r in range(_R):
                        rv = jnp.full((_L,), r, jnp.int32)
                        a = plsc.load_gather(xbuf, [rv, ja])
                        b = plsc.load_gather(xbuf, [rv, jb])
                        obuf[r, pl.ds(off, _L)] = (
                            c0 + a * (c1 + c3 * b) + c2 * b)

                o_dma(g, phase).start()
            return carry

        lax.fori_loop(0, ngroups // 2, outer, 0)
        o_dma(ngroups - 2, 0).wait()
        o_dma(ngroups - 1, 1).wait()

    return sc_kernel


def kernel(x, weights, idx_a, idx_b):
    batch, in_dim = x.shape
    out_dim = weights.shape[0]
    c, pk = _prep(weights, idx_a.astype(jnp.int32), idx_b.astype(jnp.int32))
    sc = _make_sc_kernel(batch, in_dim, out_dim)
    return sc(x, c, pk)
